# Initial kernel scaffold; baseline (speedup 1.0000x reference)
import jax
import jax.numpy as jnp
from jax.experimental import pallas as pl


def kernel(A, X):
    raise NotImplementedError("write your pallas kernel here")


# speedup vs baseline: 2.7929x; 2.7929x over previous
import functools
import math

import jax
import jax.numpy as jnp
from jax import lax
from jax.experimental import pallas as pl
from jax.experimental.pallas import tpu as pltpu
from jax.experimental.pallas import tpu_sc as plsc

N = 4096
C = 64
K = 64
B = N // K
BP = 2 * B
CK = C * K
CB = C * B

R_SEL = 256
JB_SEL = 512
NI_SEL = N // R_SEL
NJ_SEL = N // JB_SEL
JB_M = 512
R_REF = 64
NI_REF = N // R_REF
R_TAIL = 256
NI_TAIL = N // R_TAIL

NUM_CORES = 2
NUM_SUBCORES = 16
NW = NUM_CORES * NUM_SUBCORES
RPW = N // NW


def _build_m_body(g_ref, m_ref):
    gb = g_ref[...]
    iota_k = lax.broadcasted_iota(jnp.int32, (1, K), 1)
    cols = []
    for c in range(C):
        gc = gb[:, c:c + 1]
        cols.append((gc == iota_k).astype(jnp.float32))
    m_ref[...] = jnp.concatenate(cols, axis=1)


def _chunk_body(a_ref, m_ref, kst_ref, res_ref, acc_ref):
    j = pl.program_id(1)

    @pl.when(j == 0)
    def _init():
        acc_ref[...] = jnp.zeros_like(acc_ref)

    acc_ref[...] += jnp.dot(a_ref[...], m_ref[...],
                            preferred_element_type=jnp.float32)

    @pl.when(j == NJ_SEL - 1)
    def _finale():
        sb = acc_ref[...]
        tri = (lax.broadcasted_iota(jnp.int32, (K, K), 0)
               <= lax.broadcasted_iota(jnp.int32, (K, K), 1)).astype(jnp.float32)
        kst_cols, res_cols = [], []
        for c in range(C):
            sc = sb[:, c * K:(c + 1) * K]
            cs = jnp.dot(sc, tri, preferred_element_type=jnp.float32)
            halfv = cs[:, K - 1:K] * 0.5
            ltm = cs < halfv
            kst_cols.append(jnp.sum(ltm.astype(jnp.int32), axis=1, keepdims=True))
            prefix = jnp.max(jnp.where(ltm, cs, 0.0), axis=1, keepdims=True)
            res_cols.append(halfv - prefix)
        kst_ref[...] = jnp.concatenate(kst_cols, axis=1)
        res_ref[...] = jnp.concatenate(res_cols, axis=1)


def _sc_gather_body(aflat, sidx, sval, kst, wout, vout,
                    cid_v, idxc_v, valc_v, kst_v, fidx_v, aw_v,
                    sem_i, sem_v, sem_a):
    wid = lax.axis_index("s") * NUM_CORES + lax.axis_index("c")
    base_row = wid * RPW

    pltpu.sync_copy(kst.at[pl.ds(base_row * C, RPW * C)], kst_v)

    def row_body(r, carry):
        row = base_row + r
        for g4 in range(4):
            kv = kst_v[pl.ds(r * C + g4 * 16, 16)]
            cvec = lax.iota(jnp.int32, 16) + (g4 * 16)
            cid_v[pl.ds(g4 * 16, 16)] = cvec * K + kv
        cpi = pltpu.async_copy(sidx.at[cid_v], idxc_v, sem_i)
        cpv = pltpu.async_copy(sval.at[cid_v], valc_v, sem_v)
        cpi.wait()
        rown = row * N

        def c_body(c, cc):
            for g4 in range(4):
                fidx_v[pl.ds(c * B + g4 * 16, 16)] = (
                    idxc_v[c, pl.ds(g4 * 16, 16)] + rown)
            return cc

        lax.fori_loop(0, C, c_body, 0)
        cpa = pltpu.async_copy(aflat.at[fidx_v], aw_v, sem_a)
        cpv.wait()
        pltpu.sync_copy(valc_v, vout.at[pl.ds(row * C, C)])
        cpa.wait()
        pltpu.sync_copy(aw_v, wout.at[pl.ds(row * CB, CB)])
        return carry

    lax.fori_loop(0, RPW, row_body, 0)


def _refine_body(w_ref, v_ref, res_ref, xm_ref):
    w = w_ref[...]
    v = v_ref[...][:, :B]
    res = res_ref[...]
    tri = (lax.broadcasted_iota(jnp.int32, (B, B), 0)
           <= lax.broadcasted_iota(jnp.int32, (B, B), 1)).astype(jnp.float32)
    cs = jnp.dot(w, tri, preferred_element_type=jnp.float32)
    cnt = jnp.sum((cs < res).astype(jnp.int32), axis=1, keepdims=True)
    cnt = jnp.minimum(cnt, B - 1)
    iota_b = lax.broadcasted_iota(jnp.int32, (1, B), 1)
    val = jnp.sum(jnp.where(iota_b == cnt, v, 0.0), axis=1, keepdims=True)
    xm_ref[...] = val


def _tail_body(xm_ref, xt_ref, x_ref, a_ref, out_ref):
    xm = xm_ref[...]
    xt = xt_ref[...]
    g = jnp.dot(xm, xt, preferred_element_type=jnp.float32)
    mnorm = jnp.sum(xm * xm, axis=1, keepdims=True)
    xnorm = jnp.sum(xt * xt, axis=0, keepdims=True)
    sq = mnorm + xnorm - 2.0 * g
    dist = jnp.sqrt(jnp.maximum(sq, 0.0) + 1e-8)
    logit = dist * (-1.0 / math.sqrt(C))
    e = jnp.exp(logit - jnp.max(logit, axis=1, keepdims=True))
    a = a_ref[...]
    aw = e * a
    scale = (jnp.sum(a, axis=1, keepdims=True)
             / jnp.sum(aw, axis=1, keepdims=True))
    out_ref[...] = jnp.dot(aw * scale, x_ref[...],
                           preferred_element_type=jnp.float32)


def _build_m(g):
    return pl.pallas_call(
        _build_m_body,
        grid=(N // JB_M,),
        in_specs=[pl.BlockSpec((JB_M, C), lambda j: (j, 0))],
        out_specs=pl.BlockSpec((JB_M, CK), lambda j: (j, 0)),
        out_shape=jax.ShapeDtypeStruct((N, CK), jnp.float32),
    )(g)


def _chunk_select(A, M):
    return pl.pallas_call(
        _chunk_body,
        grid=(NI_SEL, NJ_SEL),
        in_specs=[
            pl.BlockSpec((R_SEL, JB_SEL), lambda i, j: (i, j)),
            pl.BlockSpec((JB_SEL, CK), lambda i, j: (j, 0)),
        ],
        out_specs=(
            pl.BlockSpec((R_SEL, C), lambda i, j: (i, 0)),
            pl.BlockSpec((R_SEL, C), lambda i, j: (i, 0)),
        ),
        out_shape=(
            jax.ShapeDtypeStruct((N, C), jnp.int32),
            jax.ShapeDtypeStruct((N, C), jnp.float32),
        ),
        scratch_shapes=[pltpu.VMEM((R_SEL, CK), jnp.float32)],
        compiler_params=pltpu.CompilerParams(
            dimension_semantics=("parallel", "arbitrary")),
    )(A, M)


def _sc_gather(aflat, sidx, sval, kst):
    mesh = plsc.VectorSubcoreMesh(core_axis_name="c", subcore_axis_name="s")
    fn = functools.partial(
        pl.kernel,
        mesh=mesh,
        out_type=(
            jax.ShapeDtypeStruct((N * CB,), jnp.float32),
            jax.ShapeDtypeStruct((N * C, BP), jnp.float32),
        ),
        scratch_types=[
            pltpu.VMEM((C,), jnp.int32),
            pltpu.VMEM((C, BP), jnp.int32),
            pltpu.VMEM((C, BP), jnp.float32),
            pltpu.VMEM((RPW * C,), jnp.int32),
            pltpu.VMEM((CB,), jnp.int32),
            pltpu.VMEM((CB,), jnp.float32),
            pltpu.SemaphoreType.DMA,
            pltpu.SemaphoreType.DMA,
            pltpu.SemaphoreType.DMA,
        ],
    )(_sc_gather_body)
    return fn(aflat, sidx, sval, kst)


def _refine(W, V, resid):
    return pl.pallas_call(
        _refine_body,
        grid=(NI_REF,),
        in_specs=[
            pl.BlockSpec((R_REF * C, B), lambda i: (i, 0)),
            pl.BlockSpec((R_REF * C, BP), lambda i: (i, 0)),
            pl.BlockSpec((R_REF * C, 1), lambda i: (i, 0)),
        ],
        out_specs=pl.BlockSpec((R_REF * C, 1), lambda i: (i, 0)),
        out_shape=jax.ShapeDtypeStruct((N * C, 1), jnp.float32),
        compiler_params=pltpu.CompilerParams(
            dimension_semantics=("parallel",)),
    )(W, V, resid)


def _tail(Xmed, XT, X, A):
    return pl.pallas_call(
        _tail_body,
        grid=(NI_TAIL,),
        in_specs=[
            pl.BlockSpec((R_TAIL, C), lambda i: (i, 0)),
            pl.BlockSpec((C, N), lambda i: (0, 0)),
            pl.BlockSpec((N, C), lambda i: (0, 0)),
            pl.BlockSpec((R_TAIL, N), lambda i: (i, 0)),
        ],
        out_specs=pl.BlockSpec((R_TAIL, C), lambda i: (i, 0)),
        out_shape=jax.ShapeDtypeStruct((N, C), jnp.float32),
        compiler_params=pltpu.CompilerParams(
            dimension_semantics=("parallel",)),
    )(Xmed, XT, X, A)


def kernel(A, X):
    sort = jnp.argsort(X, axis=0)
    sortedX = jnp.take_along_axis(X, sort, axis=0)
    g = jnp.zeros((N, C), jnp.int32).at[sort, jnp.arange(C)[None, :]].set(
        (jnp.arange(N)[:, None] // B).astype(jnp.int32))

    M = _build_m(g)
    kstar, resid = _chunk_select(A, M)

    sidx = jnp.pad(sort.T.reshape(CK, B).astype(jnp.int32), ((0, 0), (0, BP - B)))
    sval = jnp.pad(sortedX.T.reshape(CK, B), ((0, 0), (0, BP - B)))
    W_flat, V = _sc_gather(
        A.reshape(N * N),
        sidx,
        sval,
        kstar.reshape(N * C),
    )
    Xmed = _refine(W_flat.reshape(N * C, B), V,
                   resid.reshape(N * C, 1)).reshape(N, C)

    return _tail(Xmed, X.T, X, A)
